# baseline (speedup 1.0000x reference)
import jax
import jax.numpy as jnp
from jax import lax
from jax.experimental import pallas as pl
from jax.experimental.pallas import tpu as pltpu
from jax.experimental.pallas import tpu_sc as plsc

_NC = 2
_NS = 16
_NW = _NC * _NS


def kernel(indexes, table):
    num_indices = indexes.shape[0]
    dim = table.shape[1]
    b_per_w = num_indices // _NW
    idx = indexes.astype(jnp.int32).reshape(_NW, b_per_w)

    mesh = plsc.VectorSubcoreMesh(core_axis_name="c", subcore_axis_name="s")

    @jax.jit
    def run(table_arr, idx_arr):
        @pl.kernel(
            out_type=jax.ShapeDtypeStruct((num_indices, dim), table_arr.dtype),
            mesh=mesh,
            scratch_types=[
                pltpu.VMEM((b_per_w,), jnp.int32),
                pltpu.VMEM((b_per_w, dim), jnp.float32),
                pltpu.SemaphoreType.DMA,
                pltpu.SemaphoreType.DMA,
            ],
        )
        def gather_kernel(
            table_hbm, idx_hbm, out_hbm, idx_v, rows_v, isem, sem
        ):
            wid = lax.axis_index("s") * _NC + lax.axis_index("c")
            pltpu.async_copy(idx_hbm.at[wid], idx_v, isem).wait()

            @pl.loop(0, b_per_w // 16)
            def _(j):
                base = j * 16
                v16 = idx_v[pl.ds(base, 16)]
                for k in range(16):
                    pltpu.async_copy(
                        table_hbm.at[pl.ds(v16[k], 1)],
                        rows_v.at[pl.ds(base + k, 1)],
                        sem,
                    )

            pltpu.make_async_copy(
                table_hbm.at[pl.ds(0, b_per_w)], rows_v, sem
            ).wait()
            pltpu.sync_copy(rows_v, out_hbm.at[pl.ds(wid * b_per_w, b_per_w)])

        return gather_kernel(table_arr, idx_arr)

    return run(table, idx)
